# Initial kernel scaffold; baseline (speedup 1.0000x reference)
import jax
import jax.numpy as jnp
from jax.experimental import pallas as pl


def kernel(weights, occupations, par_idx, ham_idx):
    raise NotImplementedError("write your pallas kernel here")


# speedup vs baseline: 7.5499x; 7.5499x over previous
import functools

import numpy as np
import jax
import jax.numpy as jnp
from jax import lax
from jax.experimental import pallas as pl
from jax.experimental.pallas import tpu as pltpu
from jax.experimental.pallas import tpu_sc as plsc

_B, _P, _H = 8192, 256, 32
_NW = 32
_GM = 16
_TABLE_PAD = 1056
_SWEEPS = 6
_TINY = 1e-12


def _sc_build_body(table_hbm, widx_hbm, h1_hbm, h2_hbm, out_hbm,
                   table_v, widx_v, h1_v, h2_v, acc_v):
    hh = _H * _H
    mpw = _B // _NW
    wid = lax.axis_index("s") * 2 + lax.axis_index("c")
    pltpu.sync_copy(table_hbm, table_v)

    def group(g, carry):
        base_m = wid * mpw + g * _GM
        ebase = base_m * _P
        pltpu.sync_copy(widx_hbm.at[pl.ds(ebase, _GM * _P)], widx_v)
        pltpu.sync_copy(h1_hbm.at[pl.ds(ebase, _GM * _P)], h1_v)
        pltpu.sync_copy(h2_hbm.at[pl.ds(ebase, _GM * _P)], h2_v)

        def zero(k, c):
            acc_v[pl.ds(k * 16, 16)] = jnp.zeros((16,), jnp.float32)
            return c
        lax.fori_loop(0, _GM * hh // 16, zero, 0)

        def chunk(k, c):
            iv = widx_v[pl.ds(k * 16, 16)]
            v = plsc.load_gather(table_v, [iv]) * 0.5
            mofs = (k // (_P // 16)) * hh
            i1 = h1_v[pl.ds(k * 16, 16)] + mofs
            i2 = h2_v[pl.ds(k * 16, 16)] + mofs
            plsc.addupdate_scatter(acc_v, [i1], v)
            plsc.addupdate_scatter(acc_v, [i2], v)
            return c
        lax.fori_loop(0, _GM * _P // 16, chunk, 0)
        pltpu.sync_copy(acc_v, out_hbm.at[pl.ds(base_m * hh, _GM * hh)])
        return carry

    lax.fori_loop(0, mpw // _GM, group, 0)


def _build_ham_sc(table, widx, h1, h2):
    hh = _H * _H
    mesh = plsc.VectorSubcoreMesh(core_axis_name="c", subcore_axis_name="s")
    k = functools.partial(
        pl.kernel,
        mesh=mesh,
        compiler_params=pltpu.CompilerParams(needs_layout_passes=False),
        out_type=jax.ShapeDtypeStruct((_B * hh,), jnp.float32),
        scratch_types=[
            pltpu.VMEM((_TABLE_PAD,), jnp.float32),
            pltpu.VMEM((_GM * _P,), jnp.int32),
            pltpu.VMEM((_GM * _P,), jnp.int32),
            pltpu.VMEM((_GM * _P,), jnp.int32),
            pltpu.VMEM((_GM * hh,), jnp.float32),
        ],
    )(_sc_build_body)
    return k(table, widx, h1, h2)


_BIS_ITERS = 30
_PIVMIN = 1e-8


def _jacobi_body(ham_ref, occ_ref, out_ref):
    A = ham_ref[...]
    lane = lax.broadcasted_iota(jnp.int32, (1, _H), 1)

    def hh_step(k, A):
        onek = jnp.where(lane == k, 1.0, 0.0)
        x = jnp.sum(A * onek[:, None, :], axis=2)
        t = jnp.where(lane > k, x, 0.0)
        nrm2 = jnp.sum(t * t, axis=1, keepdims=True)
        piv = jnp.where(lane == k + 1, 1.0, 0.0)
        xp = jnp.sum(t * piv, axis=1, keepdims=True)
        sg = jnp.where(xp < 0, -1.0, 1.0)
        alpha = -sg * jnp.sqrt(nrm2)
        v = t - alpha * piv
        beta = jnp.sum(v * v, axis=1, keepdims=True)
        tau = jnp.where(beta > 1e-30, 2.0 / jnp.maximum(beta, 1e-30), 0.0)
        w = jnp.sum(A * v[:, None, :], axis=2)
        vw = jnp.sum(v * w, axis=1, keepdims=True)
        q = tau * w - (0.5 * tau * tau * vw) * v
        return A - v[:, :, None] * q[:, None, :] - q[:, :, None] * v[:, None, :]

    A = lax.fori_loop(0, _H - 2, hh_step, A)

    ii = lax.broadcasted_iota(jnp.int32, (_H, _H), 0)
    jj = lax.broadcasted_iota(jnp.int32, (_H, _H), 1)
    d = jnp.sum(jnp.where((ii == jj)[None], A, 0.0), axis=2)
    e = jnp.sum(jnp.where((jj == ii + 1)[None], A, 0.0), axis=2)
    eprev = jnp.where(lane == 0, 0.0, jnp.roll(e, 1, axis=1))
    r = jnp.abs(e) + jnp.abs(eprev)
    lo0 = jnp.min(d - r, axis=1, keepdims=True) + jnp.zeros_like(d)
    hi0 = jnp.max(d + r, axis=1, keepdims=True) + jnp.zeros_like(d)
    e2 = e * e
    kq = lane.astype(jnp.float32)

    def bis_step(_, carry):
        lo, hi = carry
        mid = 0.5 * (lo + hi)
        cnt = jnp.zeros_like(mid)
        qv = jnp.zeros_like(mid)
        for i in range(_H):
            di = d[:, i:i + 1]
            if i == 0:
                qv = di - mid
            else:
                qs = jnp.where(jnp.abs(qv) < _PIVMIN, -_PIVMIN, qv)
                qv = di - mid - e2[:, i - 1:i] / qs
            cnt = cnt + jnp.where(qv < 0, 1.0, 0.0)
        take_hi = cnt > kq
        return (jnp.where(take_hi, lo, mid), jnp.where(take_hi, mid, hi))

    lo, hi = lax.fori_loop(0, _BIS_ITERS, bis_step, (lo0, hi0))
    ev = 0.5 * (lo + hi)
    out_ref[...] = jnp.sum(ev * occ_ref[...], axis=1)


def _eig_tc(ham, occ):
    bm = 128
    grid = _B // bm
    return pl.pallas_call(
        _jacobi_body,
        grid=(grid,),
        in_specs=[
            pl.BlockSpec((bm, _H, _H), lambda i: (i, 0, 0)),
            pl.BlockSpec((bm, _H), lambda i: (i, 0)),
        ],
        out_specs=pl.BlockSpec((bm,), lambda i: (i,)),
        out_shape=jax.ShapeDtypeStruct((_B,), jnp.float32),
    )(ham, occ)


def kernel(weights, occupations, par_idx, ham_idx):
    table = jnp.pad(weights.reshape(-1).astype(jnp.float32),
                    (0, _TABLE_PAD - weights.size))
    pi = par_idx.astype(jnp.int32)
    widx = ((pi[..., 0] * 7 + pi[..., 1]) * 5 + pi[..., 2]) * 5 + pi[..., 3]
    r = ham_idx[..., 0].astype(jnp.int32)
    c = ham_idx[..., 1].astype(jnp.int32)
    h1 = r * _H + c
    h2 = c * _H + r
    ham_flat = _build_ham_sc(table, widx.reshape(-1), h1.reshape(-1),
                             h2.reshape(-1))
    ham = ham_flat.reshape(_B, _H, _H)
    return _eig_tc(ham, occupations.astype(jnp.float32))


# speedup vs baseline: 17.6175x; 2.3335x over previous
import functools

import numpy as np
import jax
import jax.numpy as jnp
from jax import lax
from jax.experimental import pallas as pl
from jax.experimental.pallas import tpu as pltpu
from jax.experimental.pallas import tpu_sc as plsc

_B, _P, _H = 8192, 256, 32
_NW = 32
_GM = 16
_TABLE_PAD = 1056
_SWEEPS = 6
_TINY = 1e-12


def _sc_build_body(table_hbm, widx_hbm, h1_hbm, h2_hbm, out_hbm,
                   table_v, widx_v, h1_v, h2_v, acc_v):
    hh = _H * _H
    mpw = _B // _NW
    wid = lax.axis_index("s") * 2 + lax.axis_index("c")
    pltpu.sync_copy(table_hbm, table_v)

    def group(g, carry):
        base_m = wid * mpw + g * _GM
        ebase = base_m * _P
        pltpu.sync_copy(widx_hbm.at[pl.ds(ebase, _GM * _P)], widx_v)
        pltpu.sync_copy(h1_hbm.at[pl.ds(ebase, _GM * _P)], h1_v)
        pltpu.sync_copy(h2_hbm.at[pl.ds(ebase, _GM * _P)], h2_v)

        def zero(k, c):
            acc_v[pl.ds(k * 16, 16)] = jnp.zeros((16,), jnp.float32)
            return c
        lax.fori_loop(0, _GM * hh // 16, zero, 0)

        def chunk(k, c):
            iv = widx_v[pl.ds(k * 16, 16)]
            v = plsc.load_gather(table_v, [iv]) * 0.5
            mofs = (k // (_P // 16)) * hh
            i1 = h1_v[pl.ds(k * 16, 16)] + mofs
            i2 = h2_v[pl.ds(k * 16, 16)] + mofs
            plsc.addupdate_scatter(acc_v, [i1], v)
            plsc.addupdate_scatter(acc_v, [i2], v)
            return c
        lax.fori_loop(0, _GM * _P // 16, chunk, 0)
        pltpu.sync_copy(acc_v, out_hbm.at[pl.ds(base_m * hh, _GM * hh)])
        return carry

    lax.fori_loop(0, mpw // _GM, group, 0)


def _build_ham_sc(table, widx, h1, h2):
    hh = _H * _H
    mesh = plsc.VectorSubcoreMesh(core_axis_name="c", subcore_axis_name="s")
    k = functools.partial(
        pl.kernel,
        mesh=mesh,
        compiler_params=pltpu.CompilerParams(needs_layout_passes=False),
        out_type=jax.ShapeDtypeStruct((_B * hh,), jnp.float32),
        scratch_types=[
            pltpu.VMEM((_TABLE_PAD,), jnp.float32),
            pltpu.VMEM((_GM * _P,), jnp.int32),
            pltpu.VMEM((_GM * _P,), jnp.int32),
            pltpu.VMEM((_GM * _P,), jnp.int32),
            pltpu.VMEM((_GM * hh,), jnp.float32),
        ],
    )(_sc_build_body)
    return k(table, widx, h1, h2)


_BIS_ITERS = 24
_PIVMIN = 1e-8


def _spread_lanes(x):
    idx = lax.broadcasted_iota(jnp.int32, (1, 1, _H), 2)
    for k in (1, 2, 4, 8, 16):
        lo = jnp.roll(x, -k, axis=2)
        hi = jnp.roll(x, k, axis=2)
        x = x + jnp.where((idx & k) == 0, lo, hi)
    return x


def _jacobi_body(ham_ref, occ_ref, out_ref):
    A = ham_ref[...]
    lane = lax.broadcasted_iota(jnp.int32, (1, _H), 1)

    def hh_step(k, A):
        onek = jnp.where(lane == k, 1.0, 0.0)
        x = jnp.sum(A * onek[:, None, :], axis=2)
        t = jnp.where(lane > k, x, 0.0)
        nrm2 = jnp.sum(t * t, axis=1, keepdims=True)
        piv = jnp.where(lane == k + 1, 1.0, 0.0)
        xp = jnp.sum(t * piv, axis=1, keepdims=True)
        sg = jnp.where(xp < 0, -1.0, 1.0)
        alpha = -sg * jnp.sqrt(nrm2)
        v = t - alpha * piv
        beta = jnp.sum(v * v, axis=1, keepdims=True)
        tau = jnp.where(beta > 1e-30, 2.0 / jnp.maximum(beta, 1e-30), 0.0)
        w = jnp.sum(A * v[:, None, :], axis=2)
        vw = jnp.sum(v * w, axis=1, keepdims=True)
        q = tau * w - (0.5 * tau * tau * vw) * v
        return A - v[:, :, None] * q[:, None, :] - q[:, :, None] * v[:, None, :]

    A = lax.fori_loop(0, _H - 2, hh_step, A)

    ii = lax.broadcasted_iota(jnp.int32, (_H, _H), 0)
    jj = lax.broadcasted_iota(jnp.int32, (_H, _H), 1)
    D3 = _spread_lanes(jnp.where((ii == jj)[None], A, 0.0))
    E = _spread_lanes(jnp.where((jj == ii + 1)[None], A, 0.0))
    E3 = E * E
    d = jnp.sum(jnp.where((ii == jj)[None], A, 0.0), axis=2)
    e = jnp.sum(jnp.where((jj == ii + 1)[None], A, 0.0), axis=2)
    eprev = jnp.where(lane == 0, 0.0, jnp.roll(e, 1, axis=1))
    r = jnp.abs(e) + jnp.abs(eprev)
    lo0 = jnp.min(d - r, axis=1, keepdims=True) + jnp.zeros_like(d)
    hi0 = jnp.max(d + r, axis=1, keepdims=True) + jnp.zeros_like(d)
    kq = lane.astype(jnp.float32)

    def bis_step(_, carry):
        lo, hi = carry
        mid = 0.5 * (lo + hi)
        qv = D3[:, 0, :] - mid
        cnt = jnp.where(qv < 0, 1.0, 0.0)
        for i in range(1, _H):
            qs = jnp.where(jnp.abs(qv) < _PIVMIN, -_PIVMIN, qv)
            qv = D3[:, i, :] - mid - E3[:, i - 1, :] / qs
            cnt = cnt + jnp.where(qv < 0, 1.0, 0.0)
        take_hi = cnt > kq
        return (jnp.where(take_hi, lo, mid), jnp.where(take_hi, mid, hi))

    lo, hi = lax.fori_loop(0, _BIS_ITERS, bis_step, (lo0, hi0))
    ev = 0.5 * (lo + hi)
    out_ref[...] = jnp.sum(ev * occ_ref[...], axis=1)


def _eig_tc(ham, occ):
    bm = 128
    grid = _B // bm
    return pl.pallas_call(
        _jacobi_body,
        grid=(grid,),
        in_specs=[
            pl.BlockSpec((bm, _H, _H), lambda i: (i, 0, 0)),
            pl.BlockSpec((bm, _H), lambda i: (i, 0)),
        ],
        out_specs=pl.BlockSpec((bm,), lambda i: (i,)),
        out_shape=jax.ShapeDtypeStruct((_B,), jnp.float32),
    )(ham, occ)


def kernel(weights, occupations, par_idx, ham_idx):
    table = jnp.pad(weights.reshape(-1).astype(jnp.float32),
                    (0, _TABLE_PAD - weights.size))
    pi = par_idx.astype(jnp.int32)
    widx = ((pi[..., 0] * 7 + pi[..., 1]) * 5 + pi[..., 2]) * 5 + pi[..., 3]
    r = ham_idx[..., 0].astype(jnp.int32)
    c = ham_idx[..., 1].astype(jnp.int32)
    h1 = r * _H + c
    h2 = c * _H + r
    ham_flat = _build_ham_sc(table, widx.reshape(-1), h1.reshape(-1),
                             h2.reshape(-1))
    ham = ham_flat.reshape(_B, _H, _H)
    return _eig_tc(ham, occupations.astype(jnp.float32))
